# Initial kernel scaffold; baseline (speedup 1.0000x reference)
import jax
import jax.numpy as jnp
from jax.experimental import pallas as pl


def kernel(input, embedding_matrix):
    raise NotImplementedError("write your pallas kernel here")


# speedup vs baseline: 1.5620x; 1.5620x over previous
import functools

import jax
import jax.numpy as jnp
from jax import lax
from jax.experimental import pallas as pl
from jax.experimental.pallas import tpu as pltpu
from jax.experimental.pallas import tpu_sc as plsc


@functools.lru_cache(maxsize=None)
def _make_gather(B, D, NC, NS, C):
    NW = NC * NS
    b_per_w = B // NW
    n_chunks = b_per_w // C
    mesh = plsc.VectorSubcoreMesh(core_axis_name="c", subcore_axis_name="s")

    @functools.partial(
        pl.kernel,
        mesh=mesh,
        out_type=jax.ShapeDtypeStruct((B, D), jnp.float32),
        scratch_types=[
            pltpu.VMEM((C,), jnp.int32),
            pltpu.VMEM((C, D), jnp.float32),
            pltpu.SemaphoreType.DMA,
        ],
        compiler_params=pltpu.CompilerParams(use_tc_tiling_on_sc=False),
    )
    def gather_kernel(idx_hbm, table_hbm, out_hbm, idx_v, rows_v, sem):
        wid = lax.axis_index("s") * NC + lax.axis_index("c")
        base = wid * b_per_w

        def body(g, carry):
            off = base + g * C
            pltpu.sync_copy(idx_hbm.at[pl.ds(off, C)], idx_v)
            pltpu.async_copy(table_hbm.at[idx_v], rows_v, sem).wait()
            pltpu.sync_copy(rows_v, out_hbm.at[pl.ds(off, C)])
            return carry

        lax.fori_loop(0, n_chunks, body, 0)

    return gather_kernel


def kernel(input, embedding_matrix):
    BATCH, FIELDS = input.shape
    V, D = embedding_matrix.shape
    B = BATCH * FIELDS
    info = plsc.get_sparse_core_info()
    NC, NS = info.num_cores, info.num_subcores
    idx_flat = input.reshape(B).astype(jnp.int32)
    out = _make_gather(B, D, NC, NS, 1664)(idx_flat, embedding_matrix)
    return out.reshape(BATCH, FIELDS, D)


# speedup vs baseline: 1.5766x; 1.0094x over previous
import functools

import jax
import jax.numpy as jnp
from jax import lax
from jax.experimental import pallas as pl
from jax.experimental.pallas import tpu as pltpu
from jax.experimental.pallas import tpu_sc as plsc


@functools.lru_cache(maxsize=None)
def _make_gather(B, D, NC, NS, C):
    NW = NC * NS
    b_per_w = B // NW
    n = b_per_w // C
    mesh = plsc.VectorSubcoreMesh(core_axis_name="c", subcore_axis_name="s")

    @functools.partial(
        pl.kernel,
        mesh=mesh,
        out_type=jax.ShapeDtypeStruct((B, D), jnp.float32),
        scratch_types=[
            pltpu.VMEM((2, C), jnp.int32),
            pltpu.VMEM((2, C, D), jnp.float32),
            pltpu.SemaphoreType.DMA((2,)),
            pltpu.SemaphoreType.DMA((2,)),
        ],
        compiler_params=pltpu.CompilerParams(use_tc_tiling_on_sc=False),
    )
    def gather_kernel(idx_hbm, table_hbm, out_hbm, idx_v, rows_v, gsem, ssem):
        wid = lax.axis_index("s") * NC + lax.axis_index("c")
        base = wid * b_per_w

        def load_idx(g):
            pltpu.sync_copy(idx_hbm.at[pl.ds(base + g * C, C)], idx_v.at[g % 2])

        def start_gather(g):
            return pltpu.async_copy(
                table_hbm.at[idx_v.at[g % 2]], rows_v.at[g % 2], gsem.at[g % 2]
            )

        def start_store(g):
            return pltpu.async_copy(
                rows_v.at[g % 2], out_hbm.at[pl.ds(base + g * C, C)], ssem.at[g % 2]
            )

        gathers = [None] * n
        stores = [None] * n
        load_idx(0)
        gathers[0] = start_gather(0)
        if n > 1:
            load_idx(1)
        for g in range(n):
            if g + 1 < n:
                if g >= 1:
                    stores[g - 1].wait()
                gathers[g + 1] = start_gather(g + 1)
            gathers[g].wait()
            stores[g] = start_store(g)
            if g + 2 < n:
                load_idx(g + 2)
        for g in range(max(0, n - 2), n):
            stores[g].wait()

    return gather_kernel


def kernel(input, embedding_matrix):
    BATCH, FIELDS = input.shape
    V, D = embedding_matrix.shape
    B = BATCH * FIELDS
    info = plsc.get_sparse_core_info()
    NC, NS = info.num_cores, info.num_subcores
    idx_flat = input.reshape(B).astype(jnp.int32)
    out = _make_gather(B, D, NC, NS, 1664)(idx_flat, embedding_matrix)
    return out.reshape(BATCH, FIELDS, D)
